# Initial kernel scaffold; baseline (speedup 1.0000x reference)
import jax
import jax.numpy as jnp
from jax.experimental import pallas as pl


def kernel(p_unit, threshold, values):
    raise NotImplementedError("write your pallas kernel here")


# speedup vs baseline: 277.6707x; 277.6707x over previous
import functools

import jax
import jax.numpy as jnp
from jax import lax
from jax.experimental import pallas as pl
from jax.experimental.pallas import tpu as pltpu
from jax.experimental.pallas import tpu_sc as plsc

LANES = 128
VEC = 16
NC, NS = 2, 16
NW = NC * NS
CH_ROWS = 16


def _sampler_body(vocab, n_chunks, p_hbm, t_hbm, v_hbm, out_hbm,
                  t_v, p_v, j_v, o_v, v_sh, sem):
    cid = lax.axis_index("c")
    sid = lax.axis_index("s")
    wid = cid * NS + sid
    row0 = wid * (n_chunks * CH_ROWS)

    @pl.when(sid == 0)
    def _():
        pltpu.sync_copy(v_hbm, v_sh)

    pltpu.sync_copy(t_hbm, t_v)
    plsc.subcore_barrier()

    vocab_f = jnp.float32(vocab)

    def chunk_body(g, carry):
        base = row0 + g * CH_ROWS
        pltpu.sync_copy(p_hbm.at[pl.ds(base, CH_ROWS)], p_v)

        def row_body(r, c2):
            for c in range(LANES // VEC):
                sl = pl.ds(c * VEC, VEC)
                p = p_v[r, sl] * vocab_f
                i = p.astype(jnp.int32)
                frac = p - i.astype(jnp.float32)
                t = plsc.load_gather(t_v, [i])
                j = i + i + jnp.where(t < frac, 1, 0)
                j_v[r, sl] = j
            return c2

        lax.fori_loop(0, CH_ROWS, row_body, 0)

        copies = [
            pltpu.async_copy(v_sh.at[j_v.at[r]], o_v.at[r], sem)
            for r in range(CH_ROWS)
        ]
        for cp in copies:
            cp.wait()
        pltpu.sync_copy(o_v, out_hbm.at[pl.ds(base, CH_ROWS)])
        return carry

    lax.fori_loop(0, n_chunks, chunk_body, 0)


def kernel(p_unit, threshold, values):
    batch, n_samples = p_unit.shape
    vocab = threshold.shape[0]
    total = batch * n_samples
    assert total % (NW * CH_ROWS * LANES) == 0
    n_rows = total // LANES
    n_chunks = n_rows // (NW * CH_ROWS)

    p2d = p_unit.reshape(n_rows, LANES)

    mesh = plsc.VectorSubcoreMesh(core_axis_name="c", subcore_axis_name="s")
    run = functools.partial(
        pl.kernel,
        mesh=mesh,
        compiler_params=pltpu.CompilerParams(needs_layout_passes=False),
        out_type=jax.ShapeDtypeStruct((n_rows, LANES), jnp.int32),
        scratch_types=[
            pltpu.VMEM((vocab,), jnp.float32),
            pltpu.VMEM((CH_ROWS, LANES), jnp.float32),
            pltpu.VMEM((CH_ROWS, LANES), jnp.int32),
            pltpu.VMEM((CH_ROWS, LANES), jnp.int32),
            pltpu.VMEM_SHARED((2 * vocab,), jnp.int32),
            pltpu.SemaphoreType.DMA,
        ],
    )(functools.partial(_sampler_body, vocab, n_chunks))

    out = run(p2d, threshold, values)
    return out.reshape(batch, n_samples)
